# baseline (speedup 1.0000x reference)
import jax
import jax.numpy as jnp
from jax import lax
from jax.experimental import pallas as pl
from jax.experimental.pallas import tpu as pltpu
from jax.experimental.pallas import tpu_sc as plsc

_N = 10000
_E = 320000
_D = 128

_NC = 2
_NS = 16
_NW = _NC * _NS
_C = 80
_CHUNKS = 125
_EPW = _C * _CHUNKS
_NBUF = 4
_G = 25
_GE = _G * _C
_NG = _CHUNKS // _G

_CNT_SUBS = 10


def _agg_body(x_hbm, ei_hbm, z2d_hbm,
              acc_out, cnt_out,
              idx_db, rows, ones_v, zcnt_v, acc_sh, cnt_sh,
              isem, gsem, ssem, csem):
  c = lax.axis_index("c")
  s = lax.axis_index("s")
  wid = s * _NC + c
  base = wid * _EPW

  for i in range(128 // 16):
    ones_v[pl.ds(i * 16, 16)] = jnp.ones((16,), jnp.float32)
  for i in range(1024 // 16):
    zcnt_v[pl.ds(i * 16, 16)] = jnp.zeros((16,), jnp.float32)

  def start_idx(g, p):
    pltpu.async_copy(ei_hbm.at[pl.ds(base + g * _GE, _GE)],
                     idx_db.at[pl.ds(2 * p * _GE, _GE)], isem.at[p, 0])
    pltpu.async_copy(ei_hbm.at[pl.ds(_E + base + g * _GE, _GE)],
                     idx_db.at[pl.ds((2 * p + 1) * _GE, _GE)], isem.at[p, 1])

  def wait_idx(g, p):
    pltpu.make_async_copy(ei_hbm.at[pl.ds(base + g * _GE, _GE)],
                          idx_db.at[pl.ds(2 * p * _GE, _GE)],
                          isem.at[p, 0]).wait()
    pltpu.make_async_copy(ei_hbm.at[pl.ds(_E + base + g * _GE, _GE)],
                          idx_db.at[pl.ds((2 * p + 1) * _GE, _GE)],
                          isem.at[p, 1]).wait()

  start_idx(0, 0)

  @pl.when(s < _CNT_SUBS)
  def _():
    pltpu.sync_copy(z2d_hbm.at[pl.ds(s * 1000, 1000)],
                    acc_sh.at[pl.ds(s * 1000, 1000)])
    pltpu.sync_copy(zcnt_v.at[pl.ds(0, 1000)],
                    cnt_sh.at[pl.ds(s * 1000, 1000)])

  plsc.subcore_barrier()

  def start_gather(k, b):
    p, j = (k // _G) % 2, k % _G
    pltpu.async_copy(
        x_hbm.at[idx_db.at[pl.ds(2 * p * _GE + j * _C, _C)]],
        rows.at[b], gsem.at[b])

  def wait_gather(k, b):
    p, j = (k // _G) % 2, k % _G
    pltpu.make_async_copy(
        x_hbm.at[idx_db.at[pl.ds(2 * p * _GE + j * _C, _C)]],
        rows.at[b], gsem.at[b]).wait()

  pend = [None] * _NBUF

  def process(k, b):
    p, j = (k // _G) % 2, k % _G
    wait_gather(k, b)
    sd = pltpu.async_copy(
        rows.at[b],
        acc_sh.at[idx_db.at[pl.ds((2 * p + 1) * _GE + j * _C, _C)]],
        ssem.at[b], add=True)
    cd = pltpu.async_copy(
        ones_v.at[pl.ds(0, _C)],
        cnt_sh.at[idx_db.at[pl.ds((2 * p + 1) * _GE + j * _C, _C)]],
        csem.at[b], add=True)
    pend[b] = (sd, cd)

  def drain(b):
    if pend[b] is not None:
      sd, cd = pend[b]
      sd.wait()
      cd.wait()
      pend[b] = None

  _DA = 2
  for k in range(_CHUNKS + _DA):
    kg = k
    if kg < _CHUNKS:
      if kg % _G == 0:
        wait_idx(kg // _G, (kg // _G) % 2)
      if kg % _G == _NBUF:
        g = kg // _G
        if g + 1 < _NG:
          start_idx(g + 1, (g + 1) % 2)
      drain(kg % _NBUF)
      start_gather(kg, kg % _NBUF)
    kp = k - _DA
    if kp >= 0:
      process(kp, kp % _NBUF)
  for b in range(_NBUF):
    drain(b)

  plsc.subcore_barrier()

  @pl.when(s < _CNT_SUBS)
  def _():
    pltpu.sync_copy(acc_sh.at[pl.ds(s * 1000, 1000)],
                    acc_out.at[c, pl.ds(s * 1000, 1000)])
    pltpu.sync_copy(cnt_sh.at[pl.ds(s * 1000, 1000)],
                    zcnt_v.at[pl.ds(0, 1000)])
    pltpu.sync_copy(zcnt_v.at[pl.ds(0, 1000)],
                    cnt_out.at[pl.ds(c * _N + s * 1000, 1000)])


_agg = pl.kernel(
    _agg_body,
    out_type=(
        jax.ShapeDtypeStruct((_NC, _N, _D), jnp.float32),
        jax.ShapeDtypeStruct((_NC * _N,), jnp.float32),
    ),
    mesh=plsc.VectorSubcoreMesh(
        core_axis_name="c", subcore_axis_name="s",
        num_cores=_NC, num_subcores=_NS),
    scratch_types=[
        pltpu.VMEM((4 * _GE,), jnp.int32),
        pltpu.VMEM((_NBUF, _C, _D), jnp.float32),
        pltpu.VMEM((128,), jnp.float32),
        pltpu.VMEM((1024,), jnp.float32),
        pltpu.VMEM_SHARED((_N, _D), jnp.float32),
        pltpu.VMEM_SHARED((_N,), jnp.float32),
        pltpu.SemaphoreType.DMA((2, 2)),
        pltpu.SemaphoreType.DMA((_NBUF,)),
        pltpu.SemaphoreType.DMA((_NBUF,)),
        pltpu.SemaphoreType.DMA((_NBUF,)),
    ],
)


_R = 1000


def _root_body(x_ref, wr_ref, o_ref):
  o_ref[...] = jnp.dot(x_ref[...], wr_ref[...],
                       preferred_element_type=jnp.float32)


def _root(x, W_r):
  return pl.pallas_call(
      _root_body,
      grid=(_N // _R,),
      in_specs=[
          pl.BlockSpec((_R, _D), lambda i: (i, 0)),
          pl.BlockSpec((_D, _D), lambda i: (0, 0)),
      ],
      out_specs=pl.BlockSpec((_R, _D), lambda i: (i, 0)),
      out_shape=jax.ShapeDtypeStruct((_N, _D), jnp.float32),
  )(x, W_r)


def _combine_body(acc_ref, cnt_ref, yr_ref, wl_ref, b_ref, o_ref):
  i = pl.program_id(0)
  summed = acc_ref[0] + acc_ref[1]
  cnt = jnp.maximum(cnt_ref[0, i] + cnt_ref[1, i], 1.0)
  mean = summed / cnt[:, None]
  out = (jnp.dot(mean, wl_ref[...], preferred_element_type=jnp.float32)
         + yr_ref[...] + b_ref[...])
  o_ref[...] = jnp.maximum(out, 0.0)


def _combine(acc, cnt, yr, W_l, b2d):
  nb = _N // _R
  return pl.pallas_call(
      _combine_body,
      grid=(nb,),
      in_specs=[
          pl.BlockSpec((_NC, _R, _D), lambda i: (0, i, 0)),
          pl.BlockSpec((_NC, _N // _R, _R), lambda i: (0, 0, 0)),
          pl.BlockSpec((_R, _D), lambda i: (i, 0)),
          pl.BlockSpec((_D, _D), lambda i: (0, 0)),
          pl.BlockSpec((1, _D), lambda i: (0, 0)),
      ],
      out_specs=pl.BlockSpec((_R, _D), lambda i: (i, 0)),
      out_shape=jax.ShapeDtypeStruct((_N, _D), jnp.float32),
  )(acc, cnt, yr, W_l, b2d)


def kernel(x, edge_index, W_l, W_r, b):
  z2d = jnp.zeros((_N, _D), jnp.float32)
  acc, cnt = _agg(x, edge_index.reshape(2 * _E), z2d)
  yr = _root(x, W_r)
  cnt3 = cnt.reshape(_NC, _N // _R, _R)
  return _combine(acc, cnt3, yr, W_l, b.reshape(1, _D))


# speedup vs baseline: 1.0352x; 1.0352x over previous
import jax
import jax.numpy as jnp
from jax import lax
from jax.experimental import pallas as pl
from jax.experimental.pallas import tpu as pltpu
from jax.experimental.pallas import tpu_sc as plsc

_N = 10000
_E = 320000
_D = 128

_NC = 2
_NS = 16
_NW = _NC * _NS
_C = 80
_CHUNKS = 125
_EPW = _C * _CHUNKS
_NBUF = 4
_G = 25
_GE = _G * _C
_NG = _CHUNKS // _G

_CNT_SUBS = 10


def _agg_body(x_hbm, ei_hbm, z2d_hbm,
              acc_out, cnt_out,
              idx_db, rows, ones_v, zcnt_v, acc_sh, cnt_sh,
              isem, gsem, ssem, csem):
  c = lax.axis_index("c")
  s = lax.axis_index("s")
  wid = s * _NC + c
  base = wid * _EPW

  for i in range(128 // 16):
    ones_v[pl.ds(i * 16, 16)] = jnp.ones((16,), jnp.float32)
  for i in range(1024 // 16):
    zcnt_v[pl.ds(i * 16, 16)] = jnp.zeros((16,), jnp.float32)

  def start_idx(g, p):
    pltpu.async_copy(ei_hbm.at[pl.ds(base + g * _GE, _GE)],
                     idx_db.at[pl.ds(2 * p * _GE, _GE)], isem.at[p, 0])
    pltpu.async_copy(ei_hbm.at[pl.ds(_E + base + g * _GE, _GE)],
                     idx_db.at[pl.ds((2 * p + 1) * _GE, _GE)], isem.at[p, 1])

  def wait_idx(g, p):
    pltpu.make_async_copy(ei_hbm.at[pl.ds(base + g * _GE, _GE)],
                          idx_db.at[pl.ds(2 * p * _GE, _GE)],
                          isem.at[p, 0]).wait()
    pltpu.make_async_copy(ei_hbm.at[pl.ds(_E + base + g * _GE, _GE)],
                          idx_db.at[pl.ds((2 * p + 1) * _GE, _GE)],
                          isem.at[p, 1]).wait()

  start_idx(0, 0)

  @pl.when(s < _CNT_SUBS)
  def _():
    pltpu.sync_copy(z2d_hbm.at[pl.ds(s * 1000, 1000)],
                    acc_sh.at[pl.ds(s * 1000, 1000)])
    pltpu.sync_copy(zcnt_v.at[pl.ds(0, 1000)],
                    cnt_sh.at[pl.ds(s * 1000, 1000)])

  plsc.subcore_barrier()

  def start_gather(k, b):
    p, j = (k // _G) % 2, k % _G
    pltpu.async_copy(
        x_hbm.at[idx_db.at[pl.ds(2 * p * _GE + j * _C, _C)]],
        rows.at[b], gsem.at[b])

  def wait_gather(k, b):
    p, j = (k // _G) % 2, k % _G
    pltpu.make_async_copy(
        x_hbm.at[idx_db.at[pl.ds(2 * p * _GE + j * _C, _C)]],
        rows.at[b], gsem.at[b]).wait()

  pend = [None] * _NBUF

  def process(k, b):
    p, j = (k // _G) % 2, k % _G
    wait_gather(k, b)
    sd = pltpu.async_copy(
        rows.at[b],
        acc_sh.at[idx_db.at[pl.ds((2 * p + 1) * _GE + j * _C, _C)]],
        ssem.at[b], add=True)
    cd = pltpu.async_copy(
        ones_v.at[pl.ds(0, _C)],
        cnt_sh.at[idx_db.at[pl.ds((2 * p + 1) * _GE + j * _C, _C)]],
        csem.at[b], add=True)
    pend[b] = (sd, cd)

  def drain(b):
    if pend[b] is not None:
      sd, cd = pend[b]
      sd.wait()
      cd.wait()
      pend[b] = None

  _DA = 3
  for k in range(_CHUNKS + _DA):
    kg = k
    if kg < _CHUNKS:
      if kg % _G == 0:
        wait_idx(kg // _G, (kg // _G) % 2)
      if kg % _G == _NBUF:
        g = kg // _G
        if g + 1 < _NG:
          start_idx(g + 1, (g + 1) % 2)
      drain(kg % _NBUF)
      start_gather(kg, kg % _NBUF)
    kp = k - _DA
    if kp >= 0:
      process(kp, kp % _NBUF)
  for b in range(_NBUF):
    drain(b)

  plsc.subcore_barrier()

  @pl.when(s < _CNT_SUBS)
  def _():
    pltpu.sync_copy(acc_sh.at[pl.ds(s * 1000, 1000)],
                    acc_out.at[c, pl.ds(s * 1000, 1000)])
    pltpu.sync_copy(cnt_sh.at[pl.ds(s * 1000, 1000)],
                    zcnt_v.at[pl.ds(0, 1000)])
    pltpu.sync_copy(zcnt_v.at[pl.ds(0, 1000)],
                    cnt_out.at[pl.ds(c * _N + s * 1000, 1000)])


_agg = pl.kernel(
    _agg_body,
    out_type=(
        jax.ShapeDtypeStruct((_NC, _N, _D), jnp.float32),
        jax.ShapeDtypeStruct((_NC * _N,), jnp.float32),
    ),
    mesh=plsc.VectorSubcoreMesh(
        core_axis_name="c", subcore_axis_name="s",
        num_cores=_NC, num_subcores=_NS),
    scratch_types=[
        pltpu.VMEM((4 * _GE,), jnp.int32),
        pltpu.VMEM((_NBUF, _C, _D), jnp.float32),
        pltpu.VMEM((128,), jnp.float32),
        pltpu.VMEM((1024,), jnp.float32),
        pltpu.VMEM_SHARED((_N, _D), jnp.float32),
        pltpu.VMEM_SHARED((_N,), jnp.float32),
        pltpu.SemaphoreType.DMA((2, 2)),
        pltpu.SemaphoreType.DMA((_NBUF,)),
        pltpu.SemaphoreType.DMA((_NBUF,)),
        pltpu.SemaphoreType.DMA((_NBUF,)),
    ],
)


_R = 1000


def _root_body(x_ref, wr_ref, o_ref):
  o_ref[...] = jnp.dot(x_ref[...], wr_ref[...],
                       preferred_element_type=jnp.float32)


def _root(x, W_r):
  return pl.pallas_call(
      _root_body,
      grid=(_N // _R,),
      in_specs=[
          pl.BlockSpec((_R, _D), lambda i: (i, 0)),
          pl.BlockSpec((_D, _D), lambda i: (0, 0)),
      ],
      out_specs=pl.BlockSpec((_R, _D), lambda i: (i, 0)),
      out_shape=jax.ShapeDtypeStruct((_N, _D), jnp.float32),
  )(x, W_r)


def _combine_body(acc_ref, cnt_ref, yr_ref, wl_ref, b_ref, o_ref):
  i = pl.program_id(0)
  summed = acc_ref[0] + acc_ref[1]
  cnt = jnp.maximum(cnt_ref[0, i] + cnt_ref[1, i], 1.0)
  mean = summed / cnt[:, None]
  out = (jnp.dot(mean, wl_ref[...], preferred_element_type=jnp.float32)
         + yr_ref[...] + b_ref[...])
  o_ref[...] = jnp.maximum(out, 0.0)


def _combine(acc, cnt, yr, W_l, b2d):
  nb = _N // _R
  return pl.pallas_call(
      _combine_body,
      grid=(nb,),
      in_specs=[
          pl.BlockSpec((_NC, _R, _D), lambda i: (0, i, 0)),
          pl.BlockSpec((_NC, _N // _R, _R), lambda i: (0, 0, 0)),
          pl.BlockSpec((_R, _D), lambda i: (i, 0)),
          pl.BlockSpec((_D, _D), lambda i: (0, 0)),
          pl.BlockSpec((1, _D), lambda i: (0, 0)),
      ],
      out_specs=pl.BlockSpec((_R, _D), lambda i: (i, 0)),
      out_shape=jax.ShapeDtypeStruct((_N, _D), jnp.float32),
  )(acc, cnt, yr, W_l, b2d)


def kernel(x, edge_index, W_l, W_r, b):
  z2d = jnp.zeros((_N, _D), jnp.float32)
  acc, cnt = _agg(x, edge_index.reshape(2 * _E), z2d)
  yr = _root(x, W_r)
  cnt3 = cnt.reshape(_NC, _N // _R, _R)
  return _combine(acc, cnt3, yr, W_l, b.reshape(1, _D))


# speedup vs baseline: 1.0356x; 1.0003x over previous
import jax
import jax.numpy as jnp
from jax import lax
from jax.experimental import pallas as pl
from jax.experimental.pallas import tpu as pltpu
from jax.experimental.pallas import tpu_sc as plsc

_N = 10000
_E = 320000
_D = 128

_NC = 2
_NS = 16
_NW = _NC * _NS
_C = 80
_CHUNKS = 125
_EPW = _C * _CHUNKS
_NBUF = 4
_G = 25
_GE = _G * _C
_NG = _CHUNKS // _G

_CNT_SUBS = 10


def _agg_body(x_hbm, ei_hbm, z2d_hbm,
              acc_out, cnt_out,
              idx_db, rows, ones_v, zcnt_v, acc_sh, cnt_sh,
              isem, gsem, ssem, csem):
  c = lax.axis_index("c")
  s = lax.axis_index("s")
  wid = s * _NC + c
  base = wid * _EPW

  for i in range(128 // 16):
    ones_v[pl.ds(i * 16, 16)] = jnp.ones((16,), jnp.float32)
  for i in range(1024 // 16):
    zcnt_v[pl.ds(i * 16, 16)] = jnp.zeros((16,), jnp.float32)

  def start_idx(g, p):
    pltpu.async_copy(ei_hbm.at[pl.ds(base + g * _GE, _GE)],
                     idx_db.at[pl.ds(2 * p * _GE, _GE)], isem.at[p, 0])
    pltpu.async_copy(ei_hbm.at[pl.ds(_E + base + g * _GE, _GE)],
                     idx_db.at[pl.ds((2 * p + 1) * _GE, _GE)], isem.at[p, 1])

  def wait_idx(g, p):
    pltpu.make_async_copy(ei_hbm.at[pl.ds(base + g * _GE, _GE)],
                          idx_db.at[pl.ds(2 * p * _GE, _GE)],
                          isem.at[p, 0]).wait()
    pltpu.make_async_copy(ei_hbm.at[pl.ds(_E + base + g * _GE, _GE)],
                          idx_db.at[pl.ds((2 * p + 1) * _GE, _GE)],
                          isem.at[p, 1]).wait()

  start_idx(0, 0)

  @pl.when(s < _CNT_SUBS)
  def _():
    pltpu.sync_copy(z2d_hbm.at[pl.ds(s * 1000, 1000)],
                    acc_sh.at[pl.ds(s * 1000, 1000)])
    pltpu.sync_copy(zcnt_v.at[pl.ds(0, 1000)],
                    cnt_sh.at[pl.ds(s * 1000, 1000)])

  plsc.subcore_barrier()

  def start_gather(k, b):
    p, j = (k // _G) % 2, k % _G
    pltpu.async_copy(
        x_hbm.at[idx_db.at[pl.ds(2 * p * _GE + j * _C, _C)]],
        rows.at[b], gsem.at[b])

  def wait_gather(k, b):
    p, j = (k // _G) % 2, k % _G
    pltpu.make_async_copy(
        x_hbm.at[idx_db.at[pl.ds(2 * p * _GE + j * _C, _C)]],
        rows.at[b], gsem.at[b]).wait()

  pend = [None] * _NBUF

  def process(k, b):
    p, j = (k // _G) % 2, k % _G
    wait_gather(k, b)
    sd = pltpu.async_copy(
        rows.at[b],
        acc_sh.at[idx_db.at[pl.ds((2 * p + 1) * _GE + j * _C, _C)]],
        ssem.at[b], add=True)
    cd = pltpu.async_copy(
        ones_v.at[pl.ds(0, _C)],
        cnt_sh.at[idx_db.at[pl.ds((2 * p + 1) * _GE + j * _C, _C)]],
        csem.at[b], add=True)
    pend[b] = (sd, cd)

  def drain(b):
    if pend[b] is not None:
      sd, cd = pend[b]
      sd.wait()
      cd.wait()
      pend[b] = None

  _DA = 3
  for k in range(_CHUNKS + _DA):
    kg = k
    if kg < _CHUNKS:
      if kg % _G == 0:
        wait_idx(kg // _G, (kg // _G) % 2)
      if kg % _G == _NBUF:
        g = kg // _G
        if g + 1 < _NG:
          start_idx(g + 1, (g + 1) % 2)
      drain(kg % _NBUF)
      start_gather(kg, kg % _NBUF)
    kp = k - _DA
    if kp >= 0:
      process(kp, kp % _NBUF)
  for b in range(_NBUF):
    drain(b)

  plsc.subcore_barrier()

  @pl.when(s < _CNT_SUBS)
  def _():
    pltpu.sync_copy(acc_sh.at[pl.ds(s * 1000, 1000)],
                    acc_out.at[c, pl.ds(s * 1000, 1000)])
    pltpu.sync_copy(cnt_sh.at[pl.ds(s * 1000, 1000)],
                    zcnt_v.at[pl.ds(0, 1000)])
    pltpu.sync_copy(zcnt_v.at[pl.ds(0, 1000)],
                    cnt_out.at[pl.ds(c * _N + s * 1000, 1000)])


_agg = pl.kernel(
    _agg_body,
    out_type=(
        jax.ShapeDtypeStruct((_NC, _N, _D), jnp.float32),
        jax.ShapeDtypeStruct((_NC * _N,), jnp.float32),
    ),
    mesh=plsc.VectorSubcoreMesh(
        core_axis_name="c", subcore_axis_name="s",
        num_cores=_NC, num_subcores=_NS),
    scratch_types=[
        pltpu.VMEM((4 * _GE,), jnp.int32),
        pltpu.VMEM((_NBUF, _C, _D), jnp.float32),
        pltpu.VMEM((128,), jnp.float32),
        pltpu.VMEM((1024,), jnp.float32),
        pltpu.VMEM_SHARED((_N, _D), jnp.float32),
        pltpu.VMEM_SHARED((_N,), jnp.float32),
        pltpu.SemaphoreType.DMA((2, 2)),
        pltpu.SemaphoreType.DMA((_NBUF,)),
        pltpu.SemaphoreType.DMA((_NBUF,)),
        pltpu.SemaphoreType.DMA((_NBUF,)),
    ],
)


_R = 1000


def _combine_body(acc_ref, cnt_ref, x_ref, wl_ref, wr_ref, b_ref, o_ref):
  i = pl.program_id(0)
  summed = acc_ref[0] + acc_ref[1]
  cnt = jnp.maximum(cnt_ref[0, i] + cnt_ref[1, i], 1.0)
  mean = summed / cnt[:, None]
  out = (jnp.dot(mean, wl_ref[...], preferred_element_type=jnp.float32)
         + jnp.dot(x_ref[...], wr_ref[...],
                   preferred_element_type=jnp.float32)
         + b_ref[...])
  o_ref[...] = jnp.maximum(out, 0.0)


def _combine(acc, cnt, x, W_l, W_r, b2d):
  nb = _N // _R
  return pl.pallas_call(
      _combine_body,
      grid=(nb,),
      in_specs=[
          pl.BlockSpec((_NC, _R, _D), lambda i: (0, i, 0)),
          pl.BlockSpec((_NC, _N // _R, _R), lambda i: (0, 0, 0)),
          pl.BlockSpec((_R, _D), lambda i: (i, 0)),
          pl.BlockSpec((_D, _D), lambda i: (0, 0)),
          pl.BlockSpec((_D, _D), lambda i: (0, 0)),
          pl.BlockSpec((1, _D), lambda i: (0, 0)),
      ],
      out_specs=pl.BlockSpec((_R, _D), lambda i: (i, 0)),
      out_shape=jax.ShapeDtypeStruct((_N, _D), jnp.float32),
  )(acc, cnt, x, W_l, W_r, b2d)


def kernel(x, edge_index, W_l, W_r, b):
  z2d = jnp.zeros((_N, _D), jnp.float32)
  acc, cnt = _agg(x, edge_index.reshape(2 * _E), z2d)
  cnt3 = cnt.reshape(_NC, _N // _R, _R)
  return _combine(acc, cnt3, x, W_l, W_r, b.reshape(1, _D))
